# baseline (speedup 1.0000x reference)
import functools

import jax
import jax.numpy as jnp
from jax.experimental import pallas as pl
from jax.experimental.pallas import tpu as pltpu
from jax.experimental.pallas import tpu_sc as plsc

_VOCAB = 1000
_SC_CORES = 2
_SC_SUBCORES = 16
_SC_LANES = 16
_SC_WIN = 512


def _prep_kernel(table_ref, tabt_ref, comb_ref):
    tab = table_ref[...]
    m = jnp.max(tab, axis=1, keepdims=True)
    lse = m + jnp.log(jnp.sum(jnp.exp(tab - m), axis=1, keepdims=True))
    tabt_ref[...] = tab.astype(jnp.bfloat16).T
    comb_ref[:_VOCAB, :] = -tab
    comb_ref[_VOCAB:_VOCAB + 1, :] = lse.T
    comb_ref[_VOCAB + 1:_VOCAB + 2, :] = jnp.zeros((1, _VOCAB), jnp.float32)


def _tc_kernel(idx_ref, tabt_ref, out_ref):
    nb = out_ref.shape[2]
    idx_row = idx_ref[0, 0, :]
    viota = jax.lax.broadcasted_iota(jnp.int32, (_VOCAB, nb), 0)
    onehot_t = (viota == idx_row[None, :]).astype(jnp.bfloat16)
    out_ref[0] = jnp.dot(tabt_ref[...], onehot_t,
                         preferred_element_type=jnp.float32)


def _sc_loss_partials(comb_flat, gidx):
    nidx = gidx.shape[1]
    mesh = plsc.VectorSubcoreMesh(core_axis_name="core",
                                  subcore_axis_name="subcore")

    @pl.kernel(
        out_type=jax.ShapeDtypeStruct(
            (_SC_CORES, _SC_SUBCORES, _SC_LANES), jnp.float32),
        mesh=mesh,
        scratch_types=[pltpu.VMEM((_SC_WIN,), jnp.float32),
                       pltpu.VMEM((_SC_LANES,), jnp.float32)],
    )
    def kern(comb_hbm, gidx_hbm, o_hbm, gath_vmem, acc_vmem):
        core = jax.lax.axis_index("core")
        sub = jax.lax.axis_index("subcore")
        acc_vmem[...] = jnp.zeros((_SC_LANES,), jnp.float32)

        def body(i_vmem):
            pltpu.sync_copy(comb_hbm.at[i_vmem.at[0]], gath_vmem)

            @pl.loop(0, _SC_WIN, step=_SC_LANES)
            def _(c):
                acc_vmem[...] += gath_vmem[pl.ds(c, _SC_LANES)]

        pltpu.emit_pipeline(
            body,
            grid=(nidx // _SC_WIN,),
            in_specs=[pl.BlockSpec((1, _SC_WIN), index_map=lambda i: (0, i))],
            out_specs=[],
            core_axis_name=("core", "subcore"),
            dimension_semantics=(pltpu.PARALLEL,),
        )(gidx_hbm)
        pltpu.sync_copy(acc_vmem, o_hbm.at[core, sub])

    return kern(comb_flat, gidx)


@functools.partial(jax.jit, static_argnames=())
def kernel(table, index, targets):
    b, t = index.shape
    n = b * t
    idx32 = index.astype(jnp.int32)
    idx = idx32.T.reshape(t, 1, b)

    tabt, comb = pl.pallas_call(
        _prep_kernel,
        out_shape=[
            jax.ShapeDtypeStruct((_VOCAB, _VOCAB), jnp.bfloat16),
            jax.ShapeDtypeStruct((_VOCAB + 2, _VOCAB), jnp.float32),
        ],
    )(table)

    pick_idx = (idx32 * _VOCAB + targets.astype(jnp.int32)).reshape(1, n)
    lse_idx = (idx32 + _VOCAB * _VOCAB).reshape(1, n)
    gidx = jnp.concatenate([pick_idx, lse_idx], axis=1)
    partials = _sc_loss_partials(comb.reshape((_VOCAB + 2) * _VOCAB), gidx)

    logits_t = pl.pallas_call(
        _tc_kernel,
        grid=(t,),
        in_specs=[
            pl.BlockSpec((1, 1, b), lambda i: (i, 0, 0)),
            pl.BlockSpec((_VOCAB, _VOCAB), lambda i: (0, 0)),
        ],
        out_specs=pl.BlockSpec((1, _VOCAB, b), lambda i: (i, 0, 0)),
        out_shape=jax.ShapeDtypeStruct((t, _VOCAB, b), jnp.float32),
    )(idx, tabt)

    logits = jnp.transpose(logits_t, (2, 0, 1))
    loss = jnp.sum(partials) / n
    return (logits, loss)
